# baseline (speedup 1.0000x reference)
import jax
import jax.numpy as jnp
from jax import lax
from jax.experimental import pallas as pl
from jax.experimental.pallas import tpu as pltpu
from jax.experimental.pallas import tpu_sc as plsc

NC = 2
NS = 16
LANES = 16
NW = NC * NS

B = 4096
EMB = 128
LCODES = 50
MED = 1000

BPW = B // NW
NBUF = 6
NCH = EMB // LANES


def _emb_body(dc_hbm, pc_hbm, dtab_hbm, ptab_hbm, out_hbm,
              idx_d, idx_p, rows, acc, sem):
    wid = lax.axis_index("s") * NC + lax.axis_index("c")
    base = wid * BPW
    pltpu.sync_copy(dc_hbm.at[pl.ds(base, BPW)], idx_d)
    pltpu.sync_copy(pc_hbm.at[pl.ds(base, BPW)], idx_p)

    def do_table(idx_v, tab_hbm, c0):
        for p in range(NBUF - 1):
            pltpu.async_copy(tab_hbm.at[idx_v.at[p]], rows.at[p], sem)

        def visit(v, carry):
            b = lax.rem(v, NBUF)
            pltpu.make_async_copy(
                tab_hbm.at[idx_v.at[v]], rows.at[b], sem).wait()
            nxt = v + (NBUF - 1)

            @pl.when(nxt < BPW)
            def _():
                pltpu.async_copy(
                    tab_hbm.at[idx_v.at[nxt]],
                    rows.at[lax.rem(nxt, NBUF)], sem)

            for h in range(4):
                cs = [h * 2, h * 2 + 1]
                accs = [rows[b, 0, pl.ds(c * LANES, LANES)] for c in cs]
                for i in range(1, LCODES):
                    for j, c in enumerate(cs):
                        accs[j] = accs[j] + rows[b, i,
                                                 pl.ds(c * LANES, LANES)]
                for j, c in enumerate(cs):
                    acc[v, pl.ds(c0 + c * LANES, LANES)] = accs[j]
            return carry
        lax.fori_loop(0, BPW, visit, 0)

    do_table(idx_d, dtab_hbm, 0)
    do_table(idx_p, ptab_hbm, EMB)
    pltpu.sync_copy(acc, out_hbm.at[pl.ds(base, BPW)])


_emb = pl.kernel(
    _emb_body,
    out_type=jax.ShapeDtypeStruct((B, 2 * EMB), jnp.float32),
    mesh=plsc.VectorSubcoreMesh(
        core_axis_name="c", subcore_axis_name="s",
        num_cores=NC, num_subcores=NS),
    scratch_types=[
        pltpu.VMEM((BPW, LCODES), jnp.int32),
        pltpu.VMEM((BPW, LCODES), jnp.int32),
        pltpu.VMEM((NBUF, LCODES, EMB), jnp.float32),
        pltpu.VMEM((BPW, 2 * EMB), jnp.float32),
        pltpu.SemaphoreType.DMA,
    ],
)


def _mlp_body(comb_ref, w1_ref, b1_ref, w2_ref, b2_ref, out_ref):
    x = comb_ref[...]
    h = lax.dot_general(x, w1_ref[...], (((1,), (1,)), ((), ())),
                        preferred_element_type=jnp.float32)
    h = jnp.maximum(h + b1_ref[...], 0.0)
    z = lax.dot_general(h, w2_ref[...], (((1,), (1,)), ((), ())),
                        preferred_element_type=jnp.float32)
    z = z + b2_ref[...]
    out_ref[...] = 1.0 / (1.0 + jnp.exp(-z))


_ROWS_BLK = 1024

_mlp = pl.pallas_call(
    _mlp_body,
    out_shape=jax.ShapeDtypeStruct((B, MED), jnp.float32),
    grid=(B // _ROWS_BLK,),
    in_specs=[
        pl.BlockSpec((_ROWS_BLK, 2 * EMB), lambda i: (i, 0)),
        pl.BlockSpec((64, 2 * EMB), lambda i: (0, 0)),
        pl.BlockSpec((1, 64), lambda i: (0, 0)),
        pl.BlockSpec((MED, 64), lambda i: (0, 0)),
        pl.BlockSpec((1, MED), lambda i: (0, 0)),
    ],
    out_specs=pl.BlockSpec((_ROWS_BLK, MED), lambda i: (i, 0)),
)


@jax.jit
def kernel(diag_codes, proc_codes, diag_table, proc_table, W1, b1, W2, b2):
    comb = _emb(diag_codes, proc_codes, diag_table, proc_table)
    return _mlp(comb, W1, b1.reshape(1, 64), W2, b2.reshape(1, MED))
